# baseline (speedup 1.0000x reference)
import functools
import numpy as np
import jax
import jax.numpy as jnp
from jax import lax
from jax.experimental import pallas as pl
from jax.experimental.pallas import tpu as pltpu
from jax.experimental.pallas import tpu_sc as plsc

K = 256
ROWS = 128
COLS = 32768
L = 16
NWORKERS = 32
ROWS_PER_W = ROWS // NWORKERS
NV_FULL = COLS // L
INT_MIN = np.int32(-2147483648)
MAG = np.int32(0x7FFFFFFF)


def _iota():
    return lax.iota(jnp.int32, L)


def _ub_from_f32(v):
    b = plsc.bitcast(v, jnp.int32)
    s = lax.shift_right_arithmetic(b, 31)
    return b ^ (s | INT_MIN)


def _sc_body(x_hbm, o_hbm, row_v, cand_a, cand_b, hist_v, tot_v, suf_v, sem_in, sem_out):
    wid = lax.axis_index("s") * 2 + lax.axis_index("c")
    lane_base = _iota() * jnp.int32(257)
    ones = jnp.ones((L,), jnp.int32)
    zeros16 = jnp.zeros((L,), jnp.int32)

    @plsc.parallel_loop(0, 4112 // L, unroll=8)
    def _(j):
        hist_v[pl.ds(j * L, L)] = zeros16

    suf_v[pl.ds(256, L)] = zeros16

    def load_ub(src_ref, kind, j):
        vec = src_ref[pl.ds(j * L, L)]
        if kind == "f32x":
            return _ub_from_f32(vec)
        if kind == "f32bits":
            return plsc.bitcast(vec, jnp.int32)
        return vec

    def digit(ub, shift):
        d = lax.shift_right_logical(ub, jnp.int32(shift))
        return d if shift == 24 else d & jnp.int32(0xFF)

    def hist_pass(src_ref, kind, nv, n, shift, full):
        @plsc.parallel_loop(0, nv, unroll=16 if full else 8)
        def _(j):
            ub = load_ub(src_ref, kind, j)
            idx = lane_base + digit(ub, shift)
            if full:
                plsc.addupdate_scatter(hist_v, [idx], ones)
            else:
                valid = (j * L + _iota()) < n
                plsc.addupdate_scatter(hist_v, [idx], ones, mask=valid)

    def totals_and_search(kneed):
        @plsc.parallel_loop(0, 256 // L, unroll=2)
        def _(j):
            def tl(lane, acc):
                sl = pl.ds(lane * 257 + j * L, L)
                acc = acc + hist_v[sl]
                hist_v[sl] = zeros16
                return acc

            tot_v[pl.ds(j * L, L)] = lax.fori_loop(0, L, tl, zeros16, unroll=True)

        carry = jnp.int32(0)
        nb = zeros16
        for jj in range(15, -1, -1):
            t = tot_v[pl.ds(jj * L, L)]
            c = plsc.cumsum(lax.rev(t, (0,))) + carry
            s = lax.rev(c, (0,))
            suf_v[pl.ds(jj * L, L)] = s
            nb = nb + plsc.all_reduce_population_count(s >= kneed)
            carry = c[15]

        bsel = nb[0] - 1
        kneed_next = kneed - suf_v[pl.ds(bsel + 1, L)][0]
        return bsel, kneed_next, tot_v[pl.ds(bsel, L)][0]

    def filter_pass(src_ref, kind, dst_ref, dst_f32, nv, n, shift, bsel, full):
        def body(j, off):
            ub = load_ub(src_ref, kind, j)
            m = digit(ub, shift) == bsel
            if not full:
                m = ((j * L + _iota()) < n) & m
            pos = off + plsc.cumsum(m.astype(jnp.int32))
            val = plsc.bitcast(ub, jnp.float32) if dst_f32 else ub
            plsc.store_scatter(dst_ref, [pos], val, mask=m)
            return off + plsc.all_reduce_population_count(m)

        plsc.parallel_loop(0, nv, carry=zeros16 - 1, unroll=16)(body)

    first = wid * ROWS_PER_W
    pltpu.async_copy(x_hbm.at[first], row_v.at[pl.ds(0, COLS)], sem_in)

    def process_row(r, in_ref, a_ref):
        pltpu.make_async_copy(x_hbm.at[r], in_ref.at[pl.ds(0, COLS)], sem_in).wait()

        hist_pass(in_ref, "f32x", NV_FULL, COLS, 24, True)
        b0, k1, n1 = totals_and_search(jnp.int32(K))
        filter_pass(in_ref, "f32x", a_ref, True, NV_FULL, COLS, 24, b0, True)

        nv1 = (n1 + (L - 1)) // L
        hist_pass(a_ref, "f32bits", nv1, n1, 16, False)
        b1, k2, n2 = totals_and_search(k1)

        @pl.when(r > first)
        def _():
            pltpu.make_async_copy(
                cand_b.at[pl.ds(0, COLS)], o_hbm.at[r - 1], sem_out
            ).wait()

        filter_pass(a_ref, "f32bits", cand_b, True, nv1, n1, 16, b1, False)

        nv2 = (n2 + (L - 1)) // L
        hist_pass(cand_b, "f32bits", nv2, n2, 8, False)
        b2, k3, n3 = totals_and_search(k2)
        filter_pass(cand_b, "f32bits", a_ref, True, nv2, n2, 8, b2, False)

        nv3 = (n3 + (L - 1)) // L
        hist_pass(a_ref, "f32bits", nv3, n3, 0, False)
        b3, _, _ = totals_and_search(k3)

        @pl.when(r + 1 < first + ROWS_PER_W)
        def _():
            pltpu.async_copy(x_hbm.at[r + 1], a_ref.at[pl.ds(0, COLS)], sem_in)

        ub_t = (b0 << 24) | (b1 << 16) | (b2 << 8) | b3
        tb = jnp.where(ub_t < 0, ub_t ^ INT_MIN, ~ub_t)
        t_f = plsc.bitcast(jnp.full((L,), tb, jnp.int32), jnp.float32)

        @plsc.parallel_loop(0, NV_FULL, unroll=16)
        def _(j):
            v = in_ref[pl.ds(j * L, L)]
            cand_b[pl.ds(j * L, L)] = jnp.where(v >= t_f, v, jnp.float32(-100.0))

        pltpu.async_copy(cand_b.at[pl.ds(0, COLS)], o_hbm.at[r], sem_out)

    def pair(h, _):
        r0 = first + h * 2
        process_row(r0, row_v, cand_a)
        process_row(r0 + 1, cand_a, row_v)
        return 0

    lax.fori_loop(0, ROWS_PER_W // 2, pair, 0)
    pltpu.make_async_copy(
        cand_b.at[pl.ds(0, COLS)], o_hbm.at[first + ROWS_PER_W - 1], sem_out
    ).wait()


def kernel(x):
    mesh = plsc.VectorSubcoreMesh(core_axis_name="c", subcore_axis_name="s")
    run = pl.kernel(
        _sc_body,
        out_type=jax.ShapeDtypeStruct((ROWS, COLS), jnp.float32),
        mesh=mesh,
        scratch_types=[
            pltpu.VMEM((COLS + L,), jnp.float32),
            pltpu.VMEM((COLS + L,), jnp.float32),
            pltpu.VMEM((COLS + L,), jnp.float32),
            pltpu.VMEM((4112,), jnp.int32),
            pltpu.VMEM((256 + L,), jnp.int32),
            pltpu.VMEM((256 + L,), jnp.int32),
            pltpu.SemaphoreType.DMA,
            pltpu.SemaphoreType.DMA,
        ],
        compiler_params=pltpu.CompilerParams(needs_layout_passes=False),
    )
    return run(x)


# speedup vs baseline: 1.0204x; 1.0204x over previous
import functools
import numpy as np
import jax
import jax.numpy as jnp
from jax import lax
from jax.experimental import pallas as pl
from jax.experimental.pallas import tpu as pltpu
from jax.experimental.pallas import tpu_sc as plsc

K = 256
ROWS = 128
COLS = 32768
L = 16
NWORKERS = 32
ROWS_PER_W = ROWS // NWORKERS
NV_FULL = COLS // L
INT_MIN = np.int32(-2147483648)
MAG = np.int32(0x7FFFFFFF)


def _iota():
    return lax.iota(jnp.int32, L)


def _ub_from_f32(v):
    b = plsc.bitcast(v, jnp.int32)
    s = lax.shift_right_arithmetic(b, 31)
    return b ^ (s | INT_MIN)


def _sc_body(x_hbm, o_hbm, row_v, cand_a, cand_b, hist_v, tot_v, suf_v, sem_in, sem_out):
    wid = lax.axis_index("s") * 2 + lax.axis_index("c")
    lane_base = _iota() * jnp.int32(257)
    ones = jnp.ones((L,), jnp.int32)
    zeros16 = jnp.zeros((L,), jnp.int32)

    @plsc.parallel_loop(0, 4112 // L, unroll=8)
    def _(j):
        hist_v[pl.ds(j * L, L)] = zeros16

    suf_v[pl.ds(256, L)] = zeros16

    def load_ub(src_ref, kind, j):
        vec = src_ref[pl.ds(j * L, L)]
        if kind == "f32x":
            return _ub_from_f32(vec)
        if kind == "f32bits":
            return plsc.bitcast(vec, jnp.int32)
        return vec

    def digit(ub, shift):
        d = lax.shift_right_logical(ub, jnp.int32(shift))
        return d if shift == 24 else d & jnp.int32(0xFF)

    def hist_pass(src_ref, kind, nv, n, shift, full):
        @plsc.parallel_loop(0, nv, unroll=16 if full else 8)
        def _(j):
            ub = load_ub(src_ref, kind, j)
            idx = lane_base + digit(ub, shift)
            if full:
                plsc.addupdate_scatter(hist_v, [idx], ones)
            else:
                valid = (j * L + _iota()) < n
                plsc.addupdate_scatter(hist_v, [idx], ones, mask=valid)

    def totals_and_search(kneed):
        @plsc.parallel_loop(0, 256 // L, unroll=2)
        def _(j):
            def tl(lane, acc):
                sl = pl.ds(lane * 257 + j * L, L)
                acc = acc + hist_v[sl]
                hist_v[sl] = zeros16
                return acc

            tot_v[pl.ds(j * L, L)] = lax.fori_loop(0, L, tl, zeros16, unroll=True)

        carry = jnp.int32(0)
        nb = zeros16
        for jj in range(15, -1, -1):
            t = tot_v[pl.ds(jj * L, L)]
            c = plsc.cumsum(lax.rev(t, (0,))) + carry
            s = lax.rev(c, (0,))
            suf_v[pl.ds(jj * L, L)] = s
            nb = nb + plsc.all_reduce_population_count(s >= kneed)
            carry = c[15]

        bsel = nb[0] - 1
        kneed_next = kneed - suf_v[pl.ds(bsel + 1, L)][0]
        return bsel, kneed_next, tot_v[pl.ds(bsel, L)][0]

    def filter_pass(src_ref, kind, dst_ref, dst_f32, nv, n, shift, bsel, full):
        def body(j, off):
            ub = load_ub(src_ref, kind, j)
            m = digit(ub, shift) == bsel
            if not full:
                m = ((j * L + _iota()) < n) & m
            pos = off + plsc.cumsum(m.astype(jnp.int32))
            val = plsc.bitcast(ub, jnp.float32) if dst_f32 else ub
            plsc.store_scatter(dst_ref, [pos], val, mask=m)
            return off + plsc.all_reduce_population_count(m)

        plsc.parallel_loop(0, nv, carry=zeros16 - 1, unroll=8)(body)

    first = wid * ROWS_PER_W
    pltpu.async_copy(x_hbm.at[first], row_v.at[pl.ds(0, COLS)], sem_in)

    def process_row(r, in_ref, a_ref):
        pltpu.make_async_copy(x_hbm.at[r], in_ref.at[pl.ds(0, COLS)], sem_in).wait()

        hist_pass(in_ref, "f32x", NV_FULL, COLS, 24, True)
        b0, k1, n1 = totals_and_search(jnp.int32(K))
        filter_pass(in_ref, "f32x", a_ref, True, NV_FULL, COLS, 24, b0, True)

        nv1 = (n1 + (L - 1)) // L
        hist_pass(a_ref, "f32bits", nv1, n1, 16, False)
        b1, k2, n2 = totals_and_search(k1)

        @pl.when(r > first)
        def _():
            pltpu.make_async_copy(
                cand_b.at[pl.ds(0, COLS)], o_hbm.at[r - 1], sem_out
            ).wait()

        filter_pass(a_ref, "f32bits", cand_b, True, nv1, n1, 16, b1, False)

        nv2 = (n2 + (L - 1)) // L
        hist_pass(cand_b, "f32bits", nv2, n2, 8, False)
        b2, k3, n3 = totals_and_search(k2)
        filter_pass(cand_b, "f32bits", a_ref, True, nv2, n2, 8, b2, False)

        nv3 = (n3 + (L - 1)) // L
        hist_pass(a_ref, "f32bits", nv3, n3, 0, False)
        b3, _, _ = totals_and_search(k3)

        @pl.when(r + 1 < first + ROWS_PER_W)
        def _():
            pltpu.async_copy(x_hbm.at[r + 1], a_ref.at[pl.ds(0, COLS)], sem_in)

        ub_t = (b0 << 24) | (b1 << 16) | (b2 << 8) | b3
        tb = jnp.where(ub_t < 0, ub_t ^ INT_MIN, ~ub_t)
        t_f = plsc.bitcast(jnp.full((L,), tb, jnp.int32), jnp.float32)

        @plsc.parallel_loop(0, NV_FULL, unroll=16)
        def _(j):
            v = in_ref[pl.ds(j * L, L)]
            cand_b[pl.ds(j * L, L)] = jnp.where(v >= t_f, v, jnp.float32(-100.0))

        pltpu.async_copy(cand_b.at[pl.ds(0, COLS)], o_hbm.at[r], sem_out)

    def pair(h, _):
        r0 = first + h * 2
        process_row(r0, row_v, cand_a)
        process_row(r0 + 1, cand_a, row_v)
        return 0

    lax.fori_loop(0, ROWS_PER_W // 2, pair, 0)
    pltpu.make_async_copy(
        cand_b.at[pl.ds(0, COLS)], o_hbm.at[first + ROWS_PER_W - 1], sem_out
    ).wait()


def kernel(x):
    mesh = plsc.VectorSubcoreMesh(core_axis_name="c", subcore_axis_name="s")
    run = pl.kernel(
        _sc_body,
        out_type=jax.ShapeDtypeStruct((ROWS, COLS), jnp.float32),
        mesh=mesh,
        scratch_types=[
            pltpu.VMEM((COLS + L,), jnp.float32),
            pltpu.VMEM((COLS + L,), jnp.float32),
            pltpu.VMEM((COLS + L,), jnp.float32),
            pltpu.VMEM((4112,), jnp.int32),
            pltpu.VMEM((256 + L,), jnp.int32),
            pltpu.VMEM((256 + L,), jnp.int32),
            pltpu.SemaphoreType.DMA,
            pltpu.SemaphoreType.DMA,
        ],
        compiler_params=pltpu.CompilerParams(needs_layout_passes=False),
    )
    return run(x)
